# Initial kernel scaffold; baseline (speedup 1.0000x reference)
import jax
import jax.numpy as jnp
from jax.experimental import pallas as pl


def kernel(exercise_node_embedding, kc_node_mebedding, adj_exercise_kc, adj_EE_view, adj_KK_view, exercise_data, exercise_respond_data, seqlen, W_att, a_att, W_EE, W_KK):
    raise NotImplementedError("write your pallas kernel here")


# speedup vs baseline: 4.6025x; 4.6025x over previous
import functools

import jax
import jax.numpy as jnp
from jax import lax
from jax.experimental import pallas as pl
from jax.experimental.pallas import tpu as pltpu
from jax.experimental.pallas import tpu_sc as plsc

N_EX = 1000
N_KC = 100
D = 128
NHEADS = 4
ALPHA = 0.2

_F32 = jnp.float32


def _dense_body(xe_ref, xk_ref, adjek_ref, adjee_ref, adjkk_ref, watt_ref,
                asrc_ref, adst_ref, wee_ref, wkk_ref,
                e_out, ce_out, ck_out):
    f32 = _F32
    xe = xe_ref[...]
    xk = xk_ref[...]

    ce_out[...] = jnp.maximum(
        jnp.dot(adjee_ref[...], jnp.dot(xe, wee_ref[...],
                                        preferred_element_type=f32),
                preferred_element_type=f32), 0.0)
    ck_out[...] = jnp.maximum(
        jnp.dot(adjkk_ref[...], jnp.dot(xk, wkk_ref[...],
                                        preferred_element_type=f32),
                preferred_element_type=f32), 0.0)

    adj = adjek_ref[...]
    ones_col = jnp.ones((N_EX, 1), f32)
    row_i = lax.broadcasted_iota(jnp.int32, (D, D), 0)
    col_i = lax.broadcasted_iota(jnp.int32, (D, D), 1)

    acc = jnp.zeros((N_EX, D), f32)
    for h in range(NHEADS):
        w = watt_ref[h]
        whe = jnp.dot(xe, w, preferred_element_type=f32)
        whk = jnp.dot(xk, w, preferred_element_type=f32)
        u = jnp.dot(whe, asrc_ref[h], preferred_element_type=f32)
        v = jnp.dot(whk, adst_ref[h], preferred_element_type=f32)
        vb = lax.dot_general(ones_col, v, (((1,), (1,)), ((), ())),
                             preferred_element_type=f32)
        e = u + vb
        e = jnp.where(e > 0, e, ALPHA * e)
        e = jnp.where(adj > 0, e, -9e15)
        m = jnp.max(e, axis=1, keepdims=True)
        p = jnp.exp(e - m)
        att = p / jnp.sum(p, axis=1, keepdims=True)
        head = jnp.dot(att, whk, preferred_element_type=f32)
        head = jnp.where(head > 0, head, jnp.exp(head) - 1.0)
        r_h = jnp.where(col_i == 32 * h + row_i // 4, 0.25, 0.0).astype(f32)
        acc = acc + jnp.dot(head, r_h, preferred_element_type=f32)
    e_out[...] = acc


_dense_call = pl.pallas_call(
    _dense_body,
    out_shape=[
        jax.ShapeDtypeStruct((N_EX, D), _F32),
        jax.ShapeDtypeStruct((N_EX, D), _F32),
        jax.ShapeDtypeStruct((N_KC, D), _F32),
    ],
)


_NW = 32
_CHUNK = 128


def _sc_gather_body(nch, te, tr, idxe, idxr, out1, out2,
                    idx_v, buf1, buf2, sem1, sem2):
    wid = lax.axis_index("c") * 16 + lax.axis_index("s")
    rows_per_w = nch * _CHUNK
    obase = wid * rows_per_w

    pltpu.sync_copy(idxe.at[pl.ds(obase, rows_per_w)], idx_v)

    def body1(j, carry):
        idx_chunk = idx_v.at[pl.ds(j * _CHUNK, _CHUNK)]
        pltpu.async_copy(te.at[idx_chunk], buf1, sem1).wait()
        pltpu.sync_copy(buf1, out1.at[pl.ds(obase + j * _CHUNK, _CHUNK)])
        return carry

    lax.fori_loop(0, nch, body1, 0)

    pltpu.sync_copy(idxr.at[pl.ds(obase, rows_per_w)], idx_v)

    def body2(j, carry):
        idx_chunk = idx_v.at[pl.ds(j * _CHUNK, _CHUNK)]
        pltpu.async_copy(tr.at[idx_chunk], buf2, sem2).wait()
        pltpu.sync_copy(buf2, out2.at[pl.ds(obase + j * _CHUNK, _CHUNK)])
        return carry

    lax.fori_loop(0, nch, body2, 0)


@functools.lru_cache(maxsize=None)
def _make_sc_gather(n_rows):
    nch = n_rows // (_NW * _CHUNK)
    mesh = plsc.VectorSubcoreMesh(core_axis_name="c", subcore_axis_name="s")
    return pl.kernel(
        functools.partial(_sc_gather_body, nch),
        out_type=[
            jax.ShapeDtypeStruct((n_rows, D), _F32),
            jax.ShapeDtypeStruct((n_rows, 2 * D), _F32),
        ],
        mesh=mesh,
        scratch_types=[
            pltpu.VMEM((nch * _CHUNK,), jnp.int32),
            pltpu.VMEM((_CHUNK, D), _F32),
            pltpu.VMEM((_CHUNK, 2 * D), _F32),
            pltpu.SemaphoreType.DMA,
            pltpu.SemaphoreType.DMA,
        ],
    )


def kernel(exercise_node_embedding, kc_node_mebedding, adj_exercise_kc,
           adj_EE_view, adj_KK_view, exercise_data, exercise_respond_data,
           seqlen, W_att, a_att, W_EE, W_KK):
    b, s = exercise_data.shape
    n_rows = b * s

    asrc = a_att[:, :D]
    adst = a_att[:, D:]

    ex_emb, contrastive_e, contrastive_k = _dense_call(
        exercise_node_embedding, kc_node_mebedding, adj_exercise_kc,
        adj_EE_view, adj_KK_view, W_att, asrc, adst, W_EE, W_KK)

    table_e = jnp.concatenate([jnp.zeros((1, D), _F32), ex_emb], axis=0)
    z = jnp.zeros_like(ex_emb)
    table_r = jnp.concatenate([
        jnp.zeros((1, 2 * D), _F32),
        jnp.concatenate([z, ex_emb], axis=1),
        jnp.concatenate([ex_emb, z], axis=1),
    ], axis=0)

    idxe = exercise_data.T.astype(jnp.int32).reshape(n_rows)
    idxr = exercise_respond_data.T.astype(jnp.int32).reshape(n_rows)

    out1, out2 = _make_sc_gather(n_rows)(table_e, table_r, idxe, idxr)

    return (out2.reshape(s, b, 2 * D),
            out1.reshape(s, b, D),
            ex_emb, contrastive_e, contrastive_k)


# speedup vs baseline: 5.1681x; 1.1229x over previous
import functools

import jax
import jax.numpy as jnp
from jax import lax
from jax.experimental import pallas as pl
from jax.experimental.pallas import tpu as pltpu
from jax.experimental.pallas import tpu_sc as plsc

N_EX = 1000
N_KC = 100
D = 128
NHEADS = 4
ALPHA = 0.2

_F32 = jnp.float32


def _dense_body(xe_ref, xk_ref, adjek_ref, adjee_ref, adjkk_ref, watt_ref,
                asrc_ref, adst_ref, wee_ref, wkk_ref,
                e_out, ce_out, ck_out):
    f32 = _F32
    xe = xe_ref[...]
    xk = xk_ref[...]

    ce_out[...] = jnp.maximum(
        jnp.dot(adjee_ref[...], jnp.dot(xe, wee_ref[...],
                                        preferred_element_type=f32),
                preferred_element_type=f32), 0.0)
    ck_out[...] = jnp.maximum(
        jnp.dot(adjkk_ref[...], jnp.dot(xk, wkk_ref[...],
                                        preferred_element_type=f32),
                preferred_element_type=f32), 0.0)

    adj = adjek_ref[...]
    ones_col = jnp.ones((N_EX, 1), f32)
    row_i = lax.broadcasted_iota(jnp.int32, (D, D), 0)
    col_i = lax.broadcasted_iota(jnp.int32, (D, D), 1)

    acc = jnp.zeros((N_EX, D), f32)
    for h in range(NHEADS):
        w = watt_ref[h]
        whe = jnp.dot(xe, w, preferred_element_type=f32)
        whk = jnp.dot(xk, w, preferred_element_type=f32)
        u = jnp.dot(whe, asrc_ref[h], preferred_element_type=f32)
        v = jnp.dot(whk, adst_ref[h], preferred_element_type=f32)
        vb = lax.dot_general(ones_col, v, (((1,), (1,)), ((), ())),
                             preferred_element_type=f32)
        e = u + vb
        e = jnp.where(e > 0, e, ALPHA * e)
        e = jnp.where(adj > 0, e, -9e15)
        m = jnp.max(e, axis=1, keepdims=True)
        p = jnp.exp(e - m)
        att = p / jnp.sum(p, axis=1, keepdims=True)
        head = jnp.dot(att, whk, preferred_element_type=f32)
        head = jnp.where(head > 0, head, jnp.exp(head) - 1.0)
        r_h = jnp.where(col_i == 32 * h + row_i // 4, 0.25, 0.0).astype(f32)
        acc = acc + jnp.dot(head, r_h, preferred_element_type=f32)
    e_out[...] = acc


_dense_call = pl.pallas_call(
    _dense_body,
    out_shape=[
        jax.ShapeDtypeStruct((N_EX, D), _F32),
        jax.ShapeDtypeStruct((N_EX, D), _F32),
        jax.ShapeDtypeStruct((N_KC, D), _F32),
    ],
)


_NW = 32
_CHUNK = 64
_SLOTS = 4


def _pipelined_lookup(tbl, out, idx_v, bufs, gsems, wsems, obase, nch):
    width = bufs.shape[-1]

    def body(i, carry):
        cps = []
        for k in range(_SLOTS):
            j = i * _SLOTS + k

            @pl.when(i > 0)
            def _():
                pltpu.make_async_copy(
                    bufs.at[k], out.at[pl.ds(obase, _CHUNK)], wsems[k]).wait()

            idx_chunk = idx_v.at[pl.ds(j * _CHUNK, _CHUNK)]
            cps.append(pltpu.async_copy(tbl.at[idx_chunk], bufs.at[k],
                                        gsems[k]))
        for k in range(_SLOTS):
            j = i * _SLOTS + k
            cps[k].wait()
            pltpu.async_copy(bufs.at[k],
                             out.at[pl.ds(obase + j * _CHUNK, _CHUNK)],
                             wsems[k])
        return carry

    lax.fori_loop(0, nch // _SLOTS, body, 0)
    for k in range(_SLOTS):
        pltpu.make_async_copy(
            bufs.at[k], out.at[pl.ds(obase, _CHUNK)], wsems[k]).wait()


def _sc_gather_body(nch, te, tr, idxe, idxr, out1, out2,
                    idx_v, bufs1, bufs2, *sems):
    gsems, wsems = sems[:_SLOTS], sems[_SLOTS:]
    wid = lax.axis_index("c") * 16 + lax.axis_index("s")
    rows_per_w = nch * _CHUNK
    obase = wid * rows_per_w

    pltpu.sync_copy(idxe.at[pl.ds(obase, rows_per_w)], idx_v)
    _pipelined_lookup(te, out1, idx_v, bufs1, gsems, wsems, obase, nch)
    pltpu.sync_copy(idxr.at[pl.ds(obase, rows_per_w)], idx_v)
    _pipelined_lookup(tr, out2, idx_v, bufs2, gsems, wsems, obase, nch)


@functools.lru_cache(maxsize=None)
def _make_sc_gather(n_rows):
    nch = n_rows // (_NW * _CHUNK)
    mesh = plsc.VectorSubcoreMesh(core_axis_name="c", subcore_axis_name="s")
    return pl.kernel(
        functools.partial(_sc_gather_body, nch),
        out_type=[
            jax.ShapeDtypeStruct((n_rows, D), _F32),
            jax.ShapeDtypeStruct((n_rows, 2 * D), _F32),
        ],
        mesh=mesh,
        scratch_types=[
            pltpu.VMEM((n_rows // _NW,), jnp.int32),
            pltpu.VMEM((_SLOTS, _CHUNK, D), _F32),
            pltpu.VMEM((_SLOTS, _CHUNK, 2 * D), _F32),
        ] + [pltpu.SemaphoreType.DMA] * (2 * _SLOTS),
    )


def kernel(exercise_node_embedding, kc_node_mebedding, adj_exercise_kc,
           adj_EE_view, adj_KK_view, exercise_data, exercise_respond_data,
           seqlen, W_att, a_att, W_EE, W_KK):
    b, s = exercise_data.shape
    n_rows = b * s

    asrc = a_att[:, :D]
    adst = a_att[:, D:]

    ex_emb, contrastive_e, contrastive_k = _dense_call(
        exercise_node_embedding, kc_node_mebedding, adj_exercise_kc,
        adj_EE_view, adj_KK_view, W_att, asrc, adst, W_EE, W_KK)

    table_e = jnp.concatenate([jnp.zeros((1, D), _F32), ex_emb], axis=0)
    z = jnp.zeros_like(ex_emb)
    table_r = jnp.concatenate([
        jnp.zeros((1, 2 * D), _F32),
        jnp.concatenate([z, ex_emb], axis=1),
        jnp.concatenate([ex_emb, z], axis=1),
    ], axis=0)

    idxe = exercise_data.T.astype(jnp.int32).reshape(n_rows)
    idxr = exercise_respond_data.T.astype(jnp.int32).reshape(n_rows)

    out1, out2 = _make_sc_gather(n_rows)(table_e, table_r, idxe, idxr)

    return (out2.reshape(s, b, 2 * D),
            out1.reshape(s, b, D),
            ex_emb, contrastive_e, contrastive_k)
